# Initial kernel scaffold; baseline (speedup 1.0000x reference)
import jax
import jax.numpy as jnp
from jax.experimental import pallas as pl


def kernel(node_embeddings, node_sign_influence, adj_matrices, Wq, bq, Wk, bk, Wv, bv, Wproj, bproj, sign_weight):
    raise NotImplementedError("write your pallas kernel here")


# speedup vs baseline: 1.9779x; 1.9779x over previous
import functools

import jax
import jax.numpy as jnp
from jax import lax
from jax.experimental import pallas as pl


def _qkv_body(emb_ref, nsi_ref, sw_ref, wq_ref, bq_ref, wk_ref, bk_ref,
              wv_ref, bv_ref, qs_ref, k_ref, v_ref):
    e = emb_ref[...]
    dn = (((1,), (1,)), ((), ()))
    q = lax.dot_general(e, wq_ref[0], dn, preferred_element_type=jnp.float32)
    q = q + bq_ref[0]
    scale = nsi_ref[...] * sw_ref[0]
    qs_ref[0] = q * scale
    k = lax.dot_general(e, wk_ref[0], dn, preferred_element_type=jnp.float32)
    k_ref[0] = k + bk_ref[0]
    v = lax.dot_general(e, wv_ref[0], dn, preferred_element_type=jnp.float32)
    v_ref[0] = v + bv_ref[0]


def _attn_body(q_ref, k_ref, v_ref, adj_ref, o_ref, *, num_heads, head_dim):
    q = q_ref[0]
    k = k_ref[0]
    v = v_ref[0]
    mask = adj_ref[0] > 0
    dn = (((1,), (1,)), ((), ()))
    for h in range(num_heads):
        sl = slice(h * head_dim, (h + 1) * head_dim)
        s = lax.dot_general(q[:, sl], k[:, sl], dn,
                            preferred_element_type=jnp.float32)
        m = jnp.max(jnp.where(mask, s, -jnp.inf), axis=1, keepdims=True)
        m = jnp.where(jnp.isfinite(m), m, 0.0)
        e = jnp.where(mask, jnp.exp(s - m), 0.0)
        ssum = jnp.sum(e, axis=1, keepdims=True) + 1e-10
        w = e / ssum
        o_ref[0, :, sl] = jnp.dot(w, v[:, sl],
                                  preferred_element_type=jnp.float32)


def _proj_body(o_ref, wp_ref, b_ref, out_ref):
    r = pl.program_id(0)
    dn = (((1,), (1,)), ((), ()))
    part = lax.dot_general(o_ref[0], wp_ref[0], dn,
                           preferred_element_type=jnp.float32)

    @pl.when(r == 0)
    def _():
        out_ref[...] = part + b_ref[...]

    @pl.when(r != 0)
    def _():
        out_ref[...] = out_ref[...] + part


def kernel(node_embeddings, node_sign_influence, adj_matrices, Wq, bq, Wk,
           bk, Wv, bv, Wproj, bproj, sign_weight):
    n, d_emb = node_embeddings.shape
    num_heads, num_rel = sign_weight.shape
    hd = Wq.shape[1]
    head_dim = hd // num_heads
    d_out = Wproj.shape[0]
    sqrt_d = float(head_dim) ** 0.5

    sw_exp = (jnp.repeat(sign_weight.T, head_dim, axis=1)
              / sqrt_d)[:, None, :]
    nsi2 = node_sign_influence[:, None]
    bq3 = bq[:, None, :]
    bk3 = bk[:, None, :]
    bv3 = bv[:, None, :]
    wp3 = Wproj.reshape(d_out, num_rel, hd).transpose(1, 0, 2)
    bproj2 = bproj[None, :]

    ch = min(1024, n)
    qs, k, v = pl.pallas_call(
        _qkv_body,
        grid=(num_rel, n // ch),
        in_specs=[
            pl.BlockSpec((ch, d_emb), lambda r, c: (c, 0)),
            pl.BlockSpec((ch, 1), lambda r, c: (c, 0)),
            pl.BlockSpec((1, 1, hd), lambda r, c: (r, 0, 0)),
            pl.BlockSpec((1, hd, d_emb), lambda r, c: (r, 0, 0)),
            pl.BlockSpec((1, 1, hd), lambda r, c: (r, 0, 0)),
            pl.BlockSpec((1, hd, d_emb), lambda r, c: (r, 0, 0)),
            pl.BlockSpec((1, 1, hd), lambda r, c: (r, 0, 0)),
            pl.BlockSpec((1, hd, d_emb), lambda r, c: (r, 0, 0)),
            pl.BlockSpec((1, 1, hd), lambda r, c: (r, 0, 0)),
        ],
        out_specs=[
            pl.BlockSpec((1, ch, hd), lambda r, c: (r, c, 0)),
            pl.BlockSpec((1, ch, hd), lambda r, c: (r, c, 0)),
            pl.BlockSpec((1, ch, hd), lambda r, c: (r, c, 0)),
        ],
        out_shape=[jax.ShapeDtypeStruct((num_rel, n, hd), jnp.float32)] * 3,
    )(node_embeddings, nsi2, sw_exp, Wq, bq3, Wk, bk3, Wv, bv3)

    bm = min(256, n)
    o = pl.pallas_call(
        functools.partial(_attn_body, num_heads=num_heads,
                          head_dim=head_dim),
        grid=(num_rel, n // bm),
        in_specs=[
            pl.BlockSpec((1, bm, hd), lambda r, i: (r, i, 0)),
            pl.BlockSpec((1, n, hd), lambda r, i: (r, 0, 0)),
            pl.BlockSpec((1, n, hd), lambda r, i: (r, 0, 0)),
            pl.BlockSpec((1, bm, n), lambda r, i: (r, i, 0)),
        ],
        out_specs=pl.BlockSpec((1, bm, hd), lambda r, i: (r, i, 0)),
        out_shape=jax.ShapeDtypeStruct((num_rel, n, hd), jnp.float32),
    )(qs, k, v, adj_matrices)

    final = pl.pallas_call(
        _proj_body,
        grid=(num_rel,),
        in_specs=[
            pl.BlockSpec((1, n, hd), lambda r: (r, 0, 0)),
            pl.BlockSpec((1, d_out, hd), lambda r: (r, 0, 0)),
            pl.BlockSpec((1, d_out), lambda r: (0, 0)),
        ],
        out_specs=pl.BlockSpec((n, d_out), lambda r: (0, 0)),
        out_shape=jax.ShapeDtypeStruct((n, d_out), jnp.float32),
    )(o, wp3, bproj2)

    return final


# speedup vs baseline: 2.1018x; 1.0626x over previous
import functools

import jax
import jax.numpy as jnp
from jax import lax
from jax.experimental import pallas as pl


def _qkv_body(emb_ref, nsi_ref, sw_ref, wq_ref, bq_ref, wk_ref, bk_ref,
              wv_ref, bv_ref, qs_ref, k_ref, v_ref):
    e = emb_ref[...]
    dn = (((1,), (1,)), ((), ()))
    q = lax.dot_general(e, wq_ref[0], dn, preferred_element_type=jnp.float32)
    q = q + bq_ref[0]
    scale = nsi_ref[...] * sw_ref[0]
    qs_ref[0] = (q * scale).astype(jnp.bfloat16)
    k = lax.dot_general(e, wk_ref[0], dn, preferred_element_type=jnp.float32)
    k_ref[0] = (k + bk_ref[0]).astype(jnp.bfloat16)
    v = lax.dot_general(e, wv_ref[0], dn, preferred_element_type=jnp.float32)
    v_ref[0] = (v + bv_ref[0]).astype(jnp.bfloat16)


def _attn_body(q_ref, k_ref, v_ref, adj_ref, o_ref, *, num_heads, head_dim):
    q = q_ref[0]
    k = k_ref[0]
    v = v_ref[0]
    mask = adj_ref[0] > 0
    dn = (((1,), (1,)), ((), ()))
    for h in range(num_heads):
        sl = slice(h * head_dim, (h + 1) * head_dim)
        s = lax.dot_general(q[:, sl], k[:, sl], dn,
                            preferred_element_type=jnp.float32)
        smask = jnp.where(mask, s, -1e30)
        m = jnp.max(smask, axis=1, keepdims=True)
        m = jnp.where(m <= -1e29, 0.0, m)
        e = jnp.exp(smask - m)
        ssum = jnp.sum(e, axis=1, keepdims=True) + 1e-10
        o = lax.dot_general(e.astype(jnp.bfloat16), v[:, sl],
                            (((1,), (0,)), ((), ())),
                            preferred_element_type=jnp.float32)
        o_ref[0, :, sl] = (o / ssum).astype(jnp.bfloat16)


def _proj_body(o_ref, wp_ref, b_ref, out_ref):
    r = pl.program_id(0)
    dn = (((1,), (1,)), ((), ()))
    part = lax.dot_general(o_ref[0], wp_ref[0], dn,
                           preferred_element_type=jnp.float32)

    @pl.when(r == 0)
    def _():
        out_ref[...] = part + b_ref[...]

    @pl.when(r != 0)
    def _():
        out_ref[...] = out_ref[...] + part


def kernel(node_embeddings, node_sign_influence, adj_matrices, Wq, bq, Wk,
           bk, Wv, bv, Wproj, bproj, sign_weight):
    n, d_emb = node_embeddings.shape
    num_heads, num_rel = sign_weight.shape
    hd = Wq.shape[1]
    head_dim = hd // num_heads
    d_out = Wproj.shape[0]
    sqrt_d = float(head_dim) ** 0.5

    sw_exp = (jnp.repeat(sign_weight.T, head_dim, axis=1)
              / sqrt_d)[:, None, :]
    nsi2 = node_sign_influence[:, None]
    bq3 = bq[:, None, :]
    bk3 = bk[:, None, :]
    bv3 = bv[:, None, :]
    wp3 = Wproj.reshape(d_out, num_rel, hd).transpose(1, 0, 2)
    wp3 = wp3.astype(jnp.bfloat16)
    bproj2 = bproj[None, :]

    ch = min(1024, n)
    qs, k, v = pl.pallas_call(
        _qkv_body,
        grid=(num_rel, n // ch),
        in_specs=[
            pl.BlockSpec((ch, d_emb), lambda r, c: (c, 0)),
            pl.BlockSpec((ch, 1), lambda r, c: (c, 0)),
            pl.BlockSpec((1, 1, hd), lambda r, c: (r, 0, 0)),
            pl.BlockSpec((1, hd, d_emb), lambda r, c: (r, 0, 0)),
            pl.BlockSpec((1, 1, hd), lambda r, c: (r, 0, 0)),
            pl.BlockSpec((1, hd, d_emb), lambda r, c: (r, 0, 0)),
            pl.BlockSpec((1, 1, hd), lambda r, c: (r, 0, 0)),
            pl.BlockSpec((1, hd, d_emb), lambda r, c: (r, 0, 0)),
            pl.BlockSpec((1, 1, hd), lambda r, c: (r, 0, 0)),
        ],
        out_specs=[
            pl.BlockSpec((1, ch, hd), lambda r, c: (r, c, 0)),
            pl.BlockSpec((1, ch, hd), lambda r, c: (r, c, 0)),
            pl.BlockSpec((1, ch, hd), lambda r, c: (r, c, 0)),
        ],
        out_shape=[jax.ShapeDtypeStruct((num_rel, n, hd), jnp.bfloat16)] * 3,
    )(node_embeddings, nsi2, sw_exp, Wq, bq3, Wk, bk3, Wv, bv3)

    bm = min(256, n)
    o = pl.pallas_call(
        functools.partial(_attn_body, num_heads=num_heads,
                          head_dim=head_dim),
        grid=(num_rel, n // bm),
        in_specs=[
            pl.BlockSpec((1, bm, hd), lambda r, i: (r, i, 0)),
            pl.BlockSpec((1, n, hd), lambda r, i: (r, 0, 0)),
            pl.BlockSpec((1, n, hd), lambda r, i: (r, 0, 0)),
            pl.BlockSpec((1, bm, n), lambda r, i: (r, i, 0)),
        ],
        out_specs=pl.BlockSpec((1, bm, hd), lambda r, i: (r, i, 0)),
        out_shape=jax.ShapeDtypeStruct((num_rel, n, hd), jnp.bfloat16),
    )(qs, k, v, adj_matrices)

    final = pl.pallas_call(
        _proj_body,
        grid=(num_rel,),
        in_specs=[
            pl.BlockSpec((1, n, hd), lambda r: (r, 0, 0)),
            pl.BlockSpec((1, d_out, hd), lambda r: (r, 0, 0)),
            pl.BlockSpec((1, d_out), lambda r: (0, 0)),
        ],
        out_specs=pl.BlockSpec((n, d_out), lambda r: (0, 0)),
        out_shape=jax.ShapeDtypeStruct((n, d_out), jnp.float32),
    )(o, wp3, bproj2)

    return final


# speedup vs baseline: 2.5177x; 1.1979x over previous
import functools

import jax
import jax.numpy as jnp
from jax import lax
from jax.experimental import pallas as pl


def _qkv_body(emb_ref, nsi_ref, sw_ref, wq_ref, bq_ref, wk_ref, bk_ref,
              wv_ref, bv_ref, qs_ref, k_ref, v_ref):
    e = emb_ref[...]
    dn = (((1,), (1,)), ((), ()))
    q = lax.dot_general(e, wq_ref[0], dn, preferred_element_type=jnp.float32)
    q = q + bq_ref[0]
    scale = nsi_ref[...] * sw_ref[0]
    qs_ref[0] = (q * scale).astype(jnp.bfloat16)
    k = lax.dot_general(e, wk_ref[0], dn, preferred_element_type=jnp.float32)
    k_ref[0] = (k + bk_ref[0]).astype(jnp.bfloat16)
    v = lax.dot_general(e, wv_ref[0], dn, preferred_element_type=jnp.float32)
    v_ref[0] = (v + bv_ref[0]).astype(jnp.bfloat16)


def _attn_body(q_ref, k_ref, v_ref, adj_ref, o_ref, *, num_heads, head_dim):
    q = q_ref[0]
    k = k_ref[0]
    v = v_ref[0]
    mask01 = jnp.minimum(adj_ref[0] * 2.0, 1.0)
    dn = (((1,), (1,)), ((), ()))
    for h in range(num_heads):
        sl = slice(h * head_dim, (h + 1) * head_dim)
        s = lax.dot_general(q[:, sl], k[:, sl], dn,
                            preferred_element_type=jnp.float32)
        m = jnp.max(s, axis=1, keepdims=True)
        e = jnp.exp(s - m) * mask01
        ssum = jnp.sum(e, axis=1, keepdims=True) + 1e-10
        o = lax.dot_general(e.astype(jnp.bfloat16), v[:, sl],
                            (((1,), (0,)), ((), ())),
                            preferred_element_type=jnp.float32)
        o_ref[0, :, sl] = (o / ssum).astype(jnp.bfloat16)


def _proj_body(o_ref, wp_ref, b_ref, out_ref):
    r = pl.program_id(0)
    dn = (((1,), (1,)), ((), ()))
    part = lax.dot_general(o_ref[0], wp_ref[0], dn,
                           preferred_element_type=jnp.float32)

    @pl.when(r == 0)
    def _():
        out_ref[...] = part + b_ref[...]

    @pl.when(r != 0)
    def _():
        out_ref[...] = out_ref[...] + part


def kernel(node_embeddings, node_sign_influence, adj_matrices, Wq, bq, Wk,
           bk, Wv, bv, Wproj, bproj, sign_weight):
    n, d_emb = node_embeddings.shape
    num_heads, num_rel = sign_weight.shape
    hd = Wq.shape[1]
    head_dim = hd // num_heads
    d_out = Wproj.shape[0]
    sqrt_d = float(head_dim) ** 0.5

    sw_exp = (jnp.repeat(sign_weight.T, head_dim, axis=1)
              / sqrt_d)[:, None, :]
    nsi2 = node_sign_influence[:, None]
    bq3 = bq[:, None, :]
    bk3 = bk[:, None, :]
    bv3 = bv[:, None, :]
    wp3 = Wproj.reshape(d_out, num_rel, hd).transpose(1, 0, 2)
    wp3 = wp3.astype(jnp.bfloat16)
    bproj2 = bproj[None, :]

    ch = min(1024, n)
    qs, k, v = pl.pallas_call(
        _qkv_body,
        grid=(num_rel, n // ch),
        in_specs=[
            pl.BlockSpec((ch, d_emb), lambda r, c: (c, 0)),
            pl.BlockSpec((ch, 1), lambda r, c: (c, 0)),
            pl.BlockSpec((1, 1, hd), lambda r, c: (r, 0, 0)),
            pl.BlockSpec((1, hd, d_emb), lambda r, c: (r, 0, 0)),
            pl.BlockSpec((1, 1, hd), lambda r, c: (r, 0, 0)),
            pl.BlockSpec((1, hd, d_emb), lambda r, c: (r, 0, 0)),
            pl.BlockSpec((1, 1, hd), lambda r, c: (r, 0, 0)),
            pl.BlockSpec((1, hd, d_emb), lambda r, c: (r, 0, 0)),
            pl.BlockSpec((1, 1, hd), lambda r, c: (r, 0, 0)),
        ],
        out_specs=[
            pl.BlockSpec((1, ch, hd), lambda r, c: (r, c, 0)),
            pl.BlockSpec((1, ch, hd), lambda r, c: (r, c, 0)),
            pl.BlockSpec((1, ch, hd), lambda r, c: (r, c, 0)),
        ],
        out_shape=[jax.ShapeDtypeStruct((num_rel, n, hd), jnp.bfloat16)] * 3,
    )(node_embeddings, nsi2, sw_exp, Wq, bq3, Wk, bk3, Wv, bv3)

    bm = min(256, n)
    o = pl.pallas_call(
        functools.partial(_attn_body, num_heads=num_heads,
                          head_dim=head_dim),
        grid=(num_rel, n // bm),
        in_specs=[
            pl.BlockSpec((1, bm, hd), lambda r, i: (r, i, 0)),
            pl.BlockSpec((1, n, hd), lambda r, i: (r, 0, 0)),
            pl.BlockSpec((1, n, hd), lambda r, i: (r, 0, 0)),
            pl.BlockSpec((1, bm, n), lambda r, i: (r, i, 0)),
        ],
        out_specs=pl.BlockSpec((1, bm, hd), lambda r, i: (r, i, 0)),
        out_shape=jax.ShapeDtypeStruct((num_rel, n, hd), jnp.bfloat16),
    )(qs, k, v, adj_matrices)

    final = pl.pallas_call(
        _proj_body,
        grid=(num_rel,),
        in_specs=[
            pl.BlockSpec((1, n, hd), lambda r: (r, 0, 0)),
            pl.BlockSpec((1, d_out, hd), lambda r: (r, 0, 0)),
            pl.BlockSpec((1, d_out), lambda r: (0, 0)),
        ],
        out_specs=pl.BlockSpec((n, d_out), lambda r: (0, 0)),
        out_shape=jax.ShapeDtypeStruct((n, d_out), jnp.float32),
    )(o, wp3, bproj2)

    return final


# speedup vs baseline: 2.7899x; 1.1081x over previous
import functools

import jax
import jax.numpy as jnp
from jax import lax
from jax.experimental import pallas as pl


def _qkv_body(emb_ref, nsi_ref, sw_ref, wq_ref, bq_ref, wk_ref, bk_ref,
              wv_ref, bv_ref, qs_ref, k_ref, v_ref):
    e = emb_ref[...]
    dn = (((1,), (1,)), ((), ()))
    q = lax.dot_general(e, wq_ref[0], dn, preferred_element_type=jnp.float32)
    q = q + bq_ref[0]
    scale = nsi_ref[...] * sw_ref[0]
    qs_ref[0] = (q * scale).astype(jnp.bfloat16)
    k = lax.dot_general(e, wk_ref[0], dn, preferred_element_type=jnp.float32)
    k_ref[0] = (k + bk_ref[0]).astype(jnp.bfloat16)
    v = lax.dot_general(e, wv_ref[0], dn, preferred_element_type=jnp.float32)
    v_ref[0] = (v + bv_ref[0]).astype(jnp.bfloat16)


def _attn_body(q_ref, k_ref, v_ref, adj_ref, o_ref, *, num_heads, head_dim):
    q = q_ref[0]
    k = k_ref[0]
    v = v_ref[0]
    mask01 = jnp.minimum(adj_ref[0] * 2.0, 1.0)
    dn = (((1,), (1,)), ((), ()))
    for h in range(num_heads):
        sl = slice(h * head_dim, (h + 1) * head_dim)
        s = lax.dot_general(q[:, sl], k[:, sl], dn,
                            preferred_element_type=jnp.float32)
        m = jnp.max(s, axis=1, keepdims=True)
        e = jnp.exp2(s - m) * mask01
        ssum = jnp.sum(e, axis=1, keepdims=True) + 1e-10
        o = lax.dot_general(e.astype(jnp.bfloat16), v[:, sl],
                            (((1,), (0,)), ((), ())),
                            preferred_element_type=jnp.float32)
        o_ref[0, :, sl] = (o / ssum).astype(jnp.bfloat16)


def _proj_body(o_ref, wp_ref, b_ref, out_ref):
    r = pl.program_id(0)
    dn = (((1,), (1,)), ((), ()))
    part = lax.dot_general(o_ref[0], wp_ref[0], dn,
                           preferred_element_type=jnp.float32)

    @pl.when(r == 0)
    def _():
        out_ref[...] = part + b_ref[...]

    @pl.when(r != 0)
    def _():
        out_ref[...] = out_ref[...] + part


def kernel(node_embeddings, node_sign_influence, adj_matrices, Wq, bq, Wk,
           bk, Wv, bv, Wproj, bproj, sign_weight):
    n, d_emb = node_embeddings.shape
    num_heads, num_rel = sign_weight.shape
    hd = Wq.shape[1]
    head_dim = hd // num_heads
    d_out = Wproj.shape[0]
    sqrt_d = float(head_dim) ** 0.5

    log2e = 1.4426950408889634
    sw_exp = (jnp.repeat(sign_weight.T, head_dim, axis=1)
              * (log2e / sqrt_d))[:, None, :]
    nsi2 = node_sign_influence[:, None]
    bq3 = bq[:, None, :]
    bk3 = bk[:, None, :]
    bv3 = bv[:, None, :]
    wp3 = Wproj.reshape(d_out, num_rel, hd).transpose(1, 0, 2)
    wp3 = wp3.astype(jnp.bfloat16)
    bproj2 = bproj[None, :]

    ch = min(1024, n)
    qs, k, v = pl.pallas_call(
        _qkv_body,
        grid=(num_rel, n // ch),
        in_specs=[
            pl.BlockSpec((ch, d_emb), lambda r, c: (c, 0)),
            pl.BlockSpec((ch, 1), lambda r, c: (c, 0)),
            pl.BlockSpec((1, 1, hd), lambda r, c: (r, 0, 0)),
            pl.BlockSpec((1, hd, d_emb), lambda r, c: (r, 0, 0)),
            pl.BlockSpec((1, 1, hd), lambda r, c: (r, 0, 0)),
            pl.BlockSpec((1, hd, d_emb), lambda r, c: (r, 0, 0)),
            pl.BlockSpec((1, 1, hd), lambda r, c: (r, 0, 0)),
            pl.BlockSpec((1, hd, d_emb), lambda r, c: (r, 0, 0)),
            pl.BlockSpec((1, 1, hd), lambda r, c: (r, 0, 0)),
        ],
        out_specs=[
            pl.BlockSpec((1, ch, hd), lambda r, c: (r, c, 0)),
            pl.BlockSpec((1, ch, hd), lambda r, c: (r, c, 0)),
            pl.BlockSpec((1, ch, hd), lambda r, c: (r, c, 0)),
        ],
        out_shape=[jax.ShapeDtypeStruct((num_rel, n, hd), jnp.bfloat16)] * 3,
    )(node_embeddings, nsi2, sw_exp, Wq, bq3, Wk, bk3, Wv, bv3)

    bm = min(512, n)
    o = pl.pallas_call(
        functools.partial(_attn_body, num_heads=num_heads,
                          head_dim=head_dim),
        grid=(num_rel, n // bm),
        in_specs=[
            pl.BlockSpec((1, bm, hd), lambda r, i: (r, i, 0)),
            pl.BlockSpec((1, n, hd), lambda r, i: (r, 0, 0)),
            pl.BlockSpec((1, n, hd), lambda r, i: (r, 0, 0)),
            pl.BlockSpec((1, bm, n), lambda r, i: (r, i, 0)),
        ],
        out_specs=pl.BlockSpec((1, bm, hd), lambda r, i: (r, i, 0)),
        out_shape=jax.ShapeDtypeStruct((num_rel, n, hd), jnp.bfloat16),
    )(qs, k, v, adj_matrices)

    final = pl.pallas_call(
        _proj_body,
        grid=(num_rel,),
        in_specs=[
            pl.BlockSpec((1, n, hd), lambda r: (r, 0, 0)),
            pl.BlockSpec((1, d_out, hd), lambda r: (r, 0, 0)),
            pl.BlockSpec((1, d_out), lambda r: (0, 0)),
        ],
        out_specs=pl.BlockSpec((n, d_out), lambda r: (0, 0)),
        out_shape=jax.ShapeDtypeStruct((n, d_out), jnp.float32),
    )(o, wp3, bproj2)

    return final
